# baseline (speedup 1.0000x reference)
import functools

import jax
import jax.numpy as jnp
from jax import lax
from jax.experimental import pallas as pl
from jax.experimental.pallas import tpu as pltpu
from jax.experimental.pallas import tpu_sc as plsc

_INFO = plsc.get_sparse_core_info()
_NC = _INFO.num_cores
_NS = _INFO.num_subcores
_NW = _NC * _NS
_L = _INFO.num_lanes

_COLS = 32768
_LPS = _COLS // 4
_KTC = 12
_KSC = 19


def _tc_relayout(mean_t, D):
    def body(in_ref, out_ref):
        y = jnp.transpose(in_ref[...])
        for s in range(4):
            out_ref[:, s * D:(s + 1) * D] = y[s * _LPS:(s + 1) * _LPS, :]

    return pl.pallas_call(
        body,
        grid=(_KTC,),
        in_specs=[pl.BlockSpec((D, _COLS), lambda i: (0, i + _KSC))],
        out_specs=pl.BlockSpec((_LPS, 128), lambda i: (i, 0)),
        out_shape=jax.ShapeDtypeStruct((_KTC * _LPS, 128), jnp.float32),
    )(mean_t)


def _sc_relayout(mean_t, V, D):
    lines = _KSC * _LPS
    lines_w = lines // _NW
    chunks = lines_w // 128

    mesh = plsc.VectorSubcoreMesh(core_axis_name="c", subcore_axis_name="s")

    @functools.partial(
        pl.kernel,
        mesh=mesh,
        out_type=jax.ShapeDtypeStruct((lines, 128), jnp.float32),
        scratch_types=[
            pltpu.VMEM((2, 4, D, 128), jnp.float32),
            pltpu.VMEM((2, 128, 128), jnp.float32),
            pltpu.SemaphoreType.DMA,
            pltpu.SemaphoreType.DMA,
        ],
        compiler_params=pltpu.CompilerParams(needs_layout_passes=False),
    )
    def relayout_kernel(mt_hbm, out_hbm, xbuf, ybuf, sin, sout):
        wid = lax.axis_index("s") * _NC + lax.axis_index("c")
        lines0 = wid * lines_w
        lanes = lax.iota(jnp.int32, _L)

        def m_of(c, s):
            gl = lines0 + c * 128
            m0 = (lax.shift_right_logical(gl, 13) * _COLS
                  + s * _LPS + (gl & (_LPS - 1)))
            return pl.multiple_of(m0, 128)

        def in_copy(c, s):
            return pltpu.make_async_copy(
                mt_hbm.at[:, pl.ds(m_of(c, s), 128)],
                xbuf.at[c & 1, s], sin)

        def out_copy(c):
            return pltpu.make_async_copy(
                ybuf.at[c & 1],
                out_hbm.at[pl.ds(lines0 + c * 128, 128)], sout)

        for s in range(4):
            in_copy(0, s).start()

        def chunk_body(c, _):
            buf = c & 1
            for s in range(4):
                in_copy(c, s).wait()

            @pl.when(c + 1 < chunks)
            def _prefetch():
                for s in range(4):
                    in_copy(c + 1, s).start()

            @pl.when(c >= 2)
            def _drain():
                out_copy(c - 2).wait()

            def gblk_body(g, _):
                gvec = g * _L + lanes
                for s in range(4):
                    for j in range(D):
                        val = xbuf[buf, s, j, pl.ds(g * _L, _L)]
                        col = jnp.full((_L,), s * D + j, jnp.int32)
                        plsc.store_scatter(ybuf.at[buf], [gvec, col], val)
                return _
            lax.fori_loop(0, 128 // _L, gblk_body, None)

            out_copy(c).start()
            return _
        lax.fori_loop(0, chunks, chunk_body, None)

        out_copy(chunks - 2).wait()
        out_copy(chunks - 1).wait()

    return relayout_kernel(mean_t)


def _sc_gather(indices, table_a, table_b, B, D):
    b_per_w = B // _NW
    o_per_w = b_per_w * D // 128
    split = _KSC * _LPS

    mesh = plsc.VectorSubcoreMesh(core_axis_name="c", subcore_axis_name="s")

    @functools.partial(
        pl.kernel,
        mesh=mesh,
        out_type=jax.ShapeDtypeStruct((B * D // 128, 128), jnp.float32),
        scratch_types=[
            pltpu.VMEM((b_per_w,), jnp.int32),
            pltpu.VMEM((b_per_w,), jnp.int32),
            pltpu.VMEM((b_per_w,), jnp.int32),
            pltpu.VMEM((b_per_w, 128), jnp.float32),
            pltpu.VMEM((o_per_w, 128), jnp.float32),
            pltpu.SemaphoreType.DMA,
        ],
        compiler_params=pltpu.CompilerParams(needs_layout_passes=False),
    )
    def gather_kernel(idx_hbm, ta_hbm, tb_hbm, out_hbm,
                      idx_v, ga_v, gb_v, rows_v, out_v, sem):
        wid = lax.axis_index("s") * _NC + lax.axis_index("c")
        base = wid * b_per_w

        pltpu.sync_copy(idx_hbm.at[pl.ds(base, b_per_w)], idx_v)

        def grp_body(i, _):
            ivec = idx_v[pl.ds(i * _L, _L)]
            line = lax.shift_left(
                lax.shift_right_logical(ivec, 15), 13) + (ivec & 8191)
            in_a = line >= split
            ga_v[pl.ds(i * _L, _L)] = jnp.where(in_a, line - split, 0)
            gb_v[pl.ds(i * _L, _L)] = jnp.where(in_a, 0, line)
            return _
        lax.fori_loop(0, b_per_w // _L, grp_body, None)

        lanes = lax.iota(jnp.int32, _L)

        def select(region_a):
            def sel_body(blk, _):
                b0 = blk * _L
                bvec = b0 + lanes
                ivec = idx_v[pl.ds(b0, _L)]
                in_a = ivec >= _KSC * _COLS
                msk = in_a if region_a else jnp.logical_not(in_a)
                src0 = lax.shift_left(
                    lax.shift_right_logical(ivec, 13) & 3, 5)
                dst0 = lax.shift_left(bvec, 5)
                for j in range(D):
                    val = plsc.load_gather(rows_v, [bvec, src0 + j])
                    flat = dst0 + j
                    plsc.store_scatter(
                        out_v,
                        [lax.shift_right_logical(flat, 7), flat & 127],
                        val,
                        mask=msk,
                    )
                return _
            lax.fori_loop(0, b_per_w // _L, sel_body, None)

        pltpu.async_copy(ta_hbm.at[ga_v], rows_v, sem).wait()
        select(True)
        pltpu.async_copy(tb_hbm.at[gb_v], rows_v, sem).wait()
        select(False)

        pltpu.sync_copy(out_v, out_hbm.at[pl.ds(wid * o_per_w, o_per_w)])

    return gather_kernel(indices, table_a, table_b)


def kernel(indices, mean, logstd):
    del logstd
    V, D = mean.shape
    B, = indices.shape
    mean_t = mean.T
    table_a = _tc_relayout(mean_t, D)
    table_b = _sc_relayout(mean_t, V, D)
    out = _sc_gather(indices.astype(jnp.int32), table_a, table_b, B, D)
    return out.reshape(B, D)


# speedup vs baseline: 3.6029x; 3.6029x over previous
import functools

import jax
import jax.numpy as jnp
from jax import lax
from jax.experimental import pallas as pl
from jax.experimental.pallas import tpu as pltpu
from jax.experimental.pallas import tpu_sc as plsc

_INFO = plsc.get_sparse_core_info()
_NC = _INFO.num_cores
_NS = _INFO.num_subcores
_NW = _NC * _NS
_L = _INFO.num_lanes


def _tc_relayout(mean_t, V, D):
    cols = 32768
    gout = cols // 4
    grid = (V + cols - 1) // cols

    def body(in_ref, out_ref):
        y = jnp.transpose(in_ref[...])
        for s in range(4):
            out_ref[:, s * D:(s + 1) * D] = y[s * gout:(s + 1) * gout, :]

    return pl.pallas_call(
        body,
        grid=(grid,),
        in_specs=[pl.BlockSpec((D, cols), lambda i: (0, i))],
        out_specs=pl.BlockSpec((gout, 128), lambda i: (i, 0)),
        out_shape=jax.ShapeDtypeStruct((grid * gout, 128), jnp.float32),
    )(mean_t)


def _sc_gather(indices, table, B, D):
    b_per_w = B // _NW
    o_per_w = b_per_w * D // 128

    mesh = plsc.VectorSubcoreMesh(core_axis_name="c", subcore_axis_name="s")

    @functools.partial(
        pl.kernel,
        mesh=mesh,
        out_type=jax.ShapeDtypeStruct((B * D // 128, 128), jnp.float32),
        scratch_types=[
            pltpu.VMEM((b_per_w,), jnp.int32),
            pltpu.VMEM((b_per_w,), jnp.int32),
            pltpu.VMEM((b_per_w, 128), jnp.float32),
            pltpu.VMEM((o_per_w, 128), jnp.float32),
            pltpu.SemaphoreType.DMA,
        ],
        compiler_params=pltpu.CompilerParams(needs_layout_passes=False),
    )
    def gather_kernel(idx_hbm, table_hbm, out_hbm,
                      idx_v, grp_v, rows_v, out_v, sem):
        wid = lax.axis_index("s") * _NC + lax.axis_index("c")
        base = wid * b_per_w

        pltpu.sync_copy(idx_hbm.at[pl.ds(base, b_per_w)], idx_v)

        def grp_body(i, _):
            ivec = idx_v[pl.ds(i * _L, _L)]
            grp_v[pl.ds(i * _L, _L)] = lax.shift_left(
                lax.shift_right_logical(ivec, 15), 13) + (ivec & 8191)
            return _
        lax.fori_loop(0, b_per_w // _L, grp_body, None)

        pltpu.async_copy(table_hbm.at[grp_v], rows_v, sem).wait()

        lanes = lax.iota(jnp.int32, _L)

        def sel_body(blk, _):
            b0 = blk * _L
            bvec = b0 + lanes
            ivec = idx_v[pl.ds(b0, _L)]
            src0 = lax.shift_left(
                lax.shift_right_logical(ivec, 13) & 3, 5)
            dst0 = lax.shift_left(bvec, 5)
            for j in range(D):
                val = plsc.load_gather(rows_v, [bvec, src0 + j])
                flat = dst0 + j
                plsc.store_scatter(
                    out_v,
                    [lax.shift_right_logical(flat, 7), flat & 127],
                    val,
                )
            return _
        lax.fori_loop(0, b_per_w // _L, sel_body, None)

        pltpu.sync_copy(out_v, out_hbm.at[pl.ds(wid * o_per_w, o_per_w)])

    return gather_kernel(indices, table)


def kernel(indices, mean, logstd):
    del logstd
    V, D = mean.shape
    B, = indices.shape
    table = _tc_relayout(mean.T, V, D)
    out = _sc_gather(indices.astype(jnp.int32), table, B, D)
    return out.reshape(B, D)
